# baseline (speedup 1.0000x reference)
import functools

import jax
import jax.numpy as jnp
from jax import lax
from jax.experimental import pallas as pl
from jax.experimental.pallas import tpu as pltpu
from jax.experimental.pallas import tpu_sc as plsc

N_CORES = 2
N_SUB = 16
LANES = 16
IDXW = 128
ROW_R = 2000


def _sc_mesh():
    return plsc.VectorSubcoreMesh(
        core_axis_name="c", subcore_axis_name="s",
        num_cores=N_CORES, num_subcores=N_SUB)


def _fill_zeros(ref, nrows):
    def body(i, _):
        ref[i] = jnp.zeros((LANES,), jnp.float32)
        return 0
    lax.fori_loop(0, nrows, body, 0)


@functools.lru_cache(maxsize=None)
def _make_deg(n_pad, e_pad):
    epw = e_pad // (N_CORES * N_SUB)
    assert epw % (8 * IDXW) == 0
    n_win = epw // (8 * IDXW)
    rps = n_pad // N_SUB
    assert rps % 8 == 0 and n_pad % N_SUB == 0

    def body(dst_hbm, out0_hbm, out1_hbm, ones_v, dstv, zbuf, acc, sem):
        c = lax.axis_index("c")
        s = lax.axis_index("s")
        def fill_ones(i, _):
            ones_v[pl.ds(i * LANES, LANES)] = jnp.ones((LANES,), jnp.float32)
            return 0
        lax.fori_loop(0, IDXW // LANES, fill_ones, 0)
        def fill_z(i, _):
            zbuf[pl.ds(i * LANES, LANES)] = jnp.zeros((LANES,), jnp.float32)
            return 0
        lax.fori_loop(0, rps // LANES, fill_z, 0)
        pltpu.sync_copy(zbuf, acc.at[pl.ds(s * rps, rps)])
        plsc.subcore_barrier()
        wid = c * N_SUB + s
        row0 = wid * (epw // IDXW)
        def win(w, _):
            pltpu.sync_copy(dst_hbm.at[pl.ds(row0 + w * 8, 8)], dstv)
            descs = [pltpu.async_copy(ones_v, acc.at[dstv.at[j]], sem, add=True)
                     for j in range(8)]
            for d in descs:
                d.wait()
            return 0
        lax.fori_loop(0, n_win, win, 0)
        plsc.subcore_barrier()
        pltpu.sync_copy(acc.at[pl.ds(s * rps, rps)], zbuf)

        @pl.when(c == 0)
        def _():
            pltpu.sync_copy(zbuf, out0_hbm.at[pl.ds(s * rps, rps)])

        @pl.when(c == 1)
        def _():
            pltpu.sync_copy(zbuf, out1_hbm.at[pl.ds(s * rps, rps)])

    return pl.kernel(
        body,
        out_type=(jax.ShapeDtypeStruct((n_pad,), jnp.float32),
                  jax.ShapeDtypeStruct((n_pad,), jnp.float32)),
        mesh=_sc_mesh(),
        scratch_types=[
            pltpu.VMEM((IDXW,), jnp.float32),
            pltpu.VMEM((8, IDXW), jnp.int32),
            pltpu.VMEM((rps,), jnp.float32),
            pltpu.VMEM_SHARED((n_pad,), jnp.float32),
            pltpu.SemaphoreType.DMA,
        ],
        compiler_params=pltpu.CompilerParams(use_tc_tiling_on_sc=False),
    )


STREAMS = 8
WCHUNK = 368


@functools.lru_cache(maxsize=None)
def _make_agg(n_blocks, n_pad, e_pad):
    eps = e_pad // N_SUB
    win_e = STREAMS * IDXW
    assert eps % win_e == 0
    n_win = eps // win_e
    rps = n_pad // N_SUB
    assert rps % 8 == 0

    def body(*refs):
        z_hbm, src_hbm, dst_hbm, out_hbm = refs[:4]
        srcv, dstv, sidx, rows, zbuf, acc, gsem, ssem = refs[4:]
        c = lax.axis_index("c")
        s = lax.axis_index("s")
        assert rps % WCHUNK == 0 and WCHUNK % 8 == 0
        _fill_zeros(zbuf, WCHUNK)

        def process(b):
            def zero(k, _):
                pltpu.sync_copy(zbuf, acc.at[pl.ds(s * rps + k * WCHUNK, WCHUNK)])
                return 0
            lax.fori_loop(0, rps // WCHUNK, zero, 0)
            plsc.subcore_barrier()
            row0 = s * (eps // IDXW)

            def win(w, _):
                r = row0 + w * STREAMS
                pltpu.sync_copy(src_hbm.at[pl.ds(r, STREAMS)], srcv)
                pltpu.sync_copy(dst_hbm.at[pl.ds(r, STREAMS)], dstv)
                for j in range(STREAMS):
                    for k in range(IDXW // LANES):
                        sl = pl.ds(k * LANES, LANES)
                        sidx[j, sl] = srcv[j, sl] * n_blocks + b
                gd = [pltpu.async_copy(z_hbm.at[sidx.at[j]],
                                       rows.at[pl.ds(j * IDXW, IDXW)], gsem)
                      for j in range(STREAMS)]
                for d in gd:
                    d.wait()
                sd = [pltpu.async_copy(rows.at[pl.ds(j * IDXW, IDXW)],
                                       acc.at[dstv.at[j]], ssem, add=True)
                      for j in range(STREAMS)]
                for d in sd:
                    d.wait()
                return 0
            lax.fori_loop(0, n_win, win, 0)
            plsc.subcore_barrier()

            def wb(k, _):
                r0 = s * rps + k * WCHUNK
                pltpu.sync_copy(acc.at[pl.ds(r0, WCHUNK)], rows.at[pl.ds(0, WCHUNK)])
                pltpu.sync_copy(rows.at[pl.ds(0, WCHUNK)],
                                out_hbm.at[pl.ds(r0, WCHUNK), b])
                return 0
            lax.fori_loop(0, rps // WCHUNK, wb, 0)
            plsc.subcore_barrier()

        for c_val in range(N_CORES):
            blocks = list(range(c_val, n_blocks, N_CORES))
            if not blocks:
                continue

            @pl.when(c == c_val)
            def _(blocks=blocks):
                for b in blocks:
                    process(b)

    return pl.kernel(
        body,
        out_type=jax.ShapeDtypeStruct((n_pad, n_blocks, LANES), jnp.float32),
        mesh=_sc_mesh(),
        scratch_types=[
            pltpu.VMEM((STREAMS, IDXW), jnp.int32),
            pltpu.VMEM((STREAMS, IDXW), jnp.int32),
            pltpu.VMEM((STREAMS, IDXW), jnp.int32),
            pltpu.VMEM((STREAMS * IDXW, LANES), jnp.float32),
            pltpu.VMEM((WCHUNK, LANES), jnp.float32),
            pltpu.VMEM_SHARED((n_pad, LANES), jnp.float32),
            pltpu.SemaphoreType.DMA,
            pltpu.SemaphoreType.DMA,
        ],
        compiler_params=pltpu.CompilerParams(use_tc_tiling_on_sc=False),
    )


def _sc_agg(z, src2, dst2, n, n_pad, e_pad):
    d = z.shape[1]
    assert d % LANES == 0
    nb = d // LANES
    zp = jnp.pad(z, ((0, n_pad - n), (0, 0))).reshape(n_pad * nb, LANES)
    y = _make_agg(nb, n_pad, e_pad)(zp, src2, dst2)
    return y.reshape(n_pad, d)[:n]


def _leaky(v):
    return jnp.where(v >= 0, v, 0.01 * v)


def _row_spec(w):
    return pl.BlockSpec((ROW_R, w), lambda i: (i, 0))


def _full_spec(shape):
    return pl.BlockSpec(shape, lambda i: (0, 0))


def _tc_call(body, n, ins, in_widths, w_shapes, out_widths):
    grid = (n // ROW_R,)
    in_specs = [_row_spec(w) for w in in_widths] + [_full_spec(s) for s in w_shapes]
    out_specs = [_row_spec(w) for w in out_widths]
    out_shape = [jax.ShapeDtypeStruct((n, w), jnp.float32) for w in out_widths]
    if len(out_widths) == 1:
        out_specs, out_shape = out_specs[0], out_shape[0]
    return pl.pallas_call(
        body, grid=grid, in_specs=in_specs, out_specs=out_specs,
        out_shape=out_shape)(*ins)


def _tc1(x, W1, d0, d1, n):
    def body(x_r, d0_r, d1_r, w_r, z_r, dinv_r):
        dinv = lax.rsqrt(d0_r[...] + d1_r[...] + 1.0)
        dinv_r[...] = dinv
        z_r[...] = dinv * jnp.dot(x_r[...], w_r[...],
                                  preferred_element_type=jnp.float32)
    return _tc_call(body, n, (x, d0, d1, W1),
                    (x.shape[1], 1, 1), (W1.shape,), (W1.shape[1], 1))


def _tc2(y1, z1, dinv, b1, n):
    def body(y_r, z_r, dv_r, b_r, o_r):
        h = _leaky(dv_r[...] * (y_r[...] + z_r[...]) + b_r[...])
        o_r[...] = dv_r[...] * h
    return _tc_call(body, n, (y1, z1, dinv, b1),
                    (y1.shape[1], z1.shape[1], 1), (b1.shape,), (y1.shape[1],))


def _tc3(y2, z2, dinv, W2, b2, W3, n):
    def body(y_r, z_r, dv_r, w2_r, b2_r, w3_r, o_r):
        t = dv_r[...] * (y_r[...] + z_r[...])
        h = _leaky(jnp.dot(t, w2_r[...], preferred_element_type=jnp.float32)
                   + b2_r[...])
        o_r[...] = dv_r[...] * jnp.dot(h, w3_r[...],
                                       preferred_element_type=jnp.float32)
    return _tc_call(body, n, (y2, z2, dinv, W2, b2, W3),
                    (y2.shape[1], z2.shape[1], 1),
                    (W2.shape, b2.shape, W3.shape), (W3.shape[1],))


def _tc4(y3, z3, dinv, b3, W4, n):
    def body(y_r, z_r, dv_r, b3_r, w4_r, o_r):
        h = _leaky(dv_r[...] * (y_r[...] + z_r[...]) + b3_r[...])
        o_r[...] = dv_r[...] * jnp.dot(h, w4_r[...],
                                       preferred_element_type=jnp.float32)
    return _tc_call(body, n, (y3, z3, dinv, b3, W4),
                    (y3.shape[1], z3.shape[1], 1),
                    (b3.shape, W4.shape), (W4.shape[1],))


def _tc5(y4, z4, dinv, b4, Wl, bl, n):
    def body(y_r, z_r, dv_r, b4_r, wl_r, bl_r, o_r):
        h = _leaky(dv_r[...] * (y_r[...] + z_r[...]) + b4_r[...])
        o_r[...] = jnp.maximum(
            jnp.dot(h, wl_r[...], preferred_element_type=jnp.float32)
            + bl_r[...], 0.0)
    return _tc_call(body, n, (y4, z4, dinv, b4, Wl, bl),
                    (y4.shape[1], z4.shape[1], 1),
                    (b4.shape, Wl.shape, bl.shape), (Wl.shape[1],))


def kernel(x, edge_index, W1, b1, W2, b2, W3, b3, W4, b4, Wl, bl):
    n = x.shape[0]
    e = edge_index.shape[1]
    assert n % ROW_R == 0

    n_pad = ((n + 127) // 128) * 128
    e_unit = N_CORES * N_SUB * STREAMS * IDXW
    e_pad = ((e + e_unit - 1) // e_unit) * e_unit

    src = edge_index[0].astype(jnp.int32)
    dst = edge_index[1].astype(jnp.int32)
    if e_pad != e:
        pad = e_pad - e
        pad_dst = n + jnp.arange(pad, dtype=jnp.int32) % (n_pad - n)
        src = jnp.concatenate([src, jnp.full((pad,), n, jnp.int32)])
        dst = jnp.concatenate([dst, pad_dst])
    src2 = src.reshape(e_pad // IDXW, IDXW)
    dst2 = dst.reshape(e_pad // IDXW, IDXW)

    deg0, deg1 = _make_deg(n_pad, e_pad)(dst2)
    d0 = deg0[:n, None]
    d1 = deg1[:n, None]

    b1r, b2r, b3r, b4r, blr = (v.reshape(1, -1) for v in (b1, b2, b3, b4, bl))

    z1, dinv = _tc1(x, W1, d0, d1, n)
    y1 = _sc_agg(z1, src2, dst2, n, n_pad, e_pad)
    z2 = _tc2(y1, z1, dinv, b1r, n)
    y2 = _sc_agg(z2, src2, dst2, n, n_pad, e_pad)
    z3 = _tc3(y2, z2, dinv, W2, b2r, W3, n)
    y3 = _sc_agg(z3, src2, dst2, n, n_pad, e_pad)
    z4 = _tc4(y3, z3, dinv, b3r, W4, n)
    y4 = _sc_agg(z4, src2, dst2, n, n_pad, e_pad)
    return _tc5(y4, z4, dinv, b4r, Wl, blr, n)


# speedup vs baseline: 1.3428x; 1.3428x over previous
import functools

import jax
import jax.numpy as jnp
from jax import lax
from jax.experimental import pallas as pl
from jax.experimental.pallas import tpu as pltpu
from jax.experimental.pallas import tpu_sc as plsc

N_CORES = 2
N_SUB = 16
LANES = 16
IDXW = 128
STREAMS = 8
WCHUNK = 368
N_PAD = 100096
ROW_R = 4352


def _sc_mesh():
    return plsc.VectorSubcoreMesh(
        core_axis_name="c", subcore_axis_name="s",
        num_cores=N_CORES, num_subcores=N_SUB)


@functools.lru_cache(maxsize=None)
def _make_deg(n_pad, e_pad):
    epw = e_pad // (N_CORES * N_SUB)
    assert epw % (8 * IDXW) == 0
    n_win = epw // (8 * IDXW)
    rps = n_pad // N_SUB
    assert rps % 8 == 0 and n_pad % N_SUB == 0

    def body(dst_hbm, out0_hbm, out1_hbm, ones_v, dstv, zbuf, acc, sem):
        c = lax.axis_index("c")
        s = lax.axis_index("s")
        def fill_ones(i, _):
            ones_v[pl.ds(i * LANES, LANES)] = jnp.ones((LANES,), jnp.float32)
            return 0
        lax.fori_loop(0, IDXW // LANES, fill_ones, 0)
        def fill_z(i, _):
            zbuf[pl.ds(i * LANES, LANES)] = jnp.zeros((LANES,), jnp.float32)
            return 0
        lax.fori_loop(0, rps // LANES, fill_z, 0)
        pltpu.sync_copy(zbuf, acc.at[pl.ds(s * rps, rps)])
        plsc.subcore_barrier()
        wid = c * N_SUB + s
        row0 = wid * (epw // IDXW)
        def win(w, _):
            pltpu.sync_copy(dst_hbm.at[pl.ds(row0 + w * 8, 8)], dstv)
            descs = [pltpu.async_copy(ones_v, acc.at[dstv.at[j]], sem, add=True)
                     for j in range(8)]
            for d in descs:
                d.wait()
            return 0
        lax.fori_loop(0, n_win, win, 0)
        plsc.subcore_barrier()
        pltpu.sync_copy(acc.at[pl.ds(s * rps, rps)], zbuf)

        @pl.when(c == 0)
        def _():
            pltpu.sync_copy(zbuf, out0_hbm.at[pl.ds(s * rps, rps)])

        @pl.when(c == 1)
        def _():
            pltpu.sync_copy(zbuf, out1_hbm.at[pl.ds(s * rps, rps)])

    return pl.kernel(
        body,
        out_type=(jax.ShapeDtypeStruct((n_pad,), jnp.float32),
                  jax.ShapeDtypeStruct((n_pad,), jnp.float32)),
        mesh=_sc_mesh(),
        scratch_types=[
            pltpu.VMEM((IDXW,), jnp.float32),
            pltpu.VMEM((8, IDXW), jnp.int32),
            pltpu.VMEM((rps,), jnp.float32),
            pltpu.VMEM_SHARED((n_pad,), jnp.float32),
            pltpu.SemaphoreType.DMA,
        ],
        compiler_params=pltpu.CompilerParams(use_tc_tiling_on_sc=False),
    )


@functools.lru_cache(maxsize=None)
def _make_agg(n_blocks, n_pad, e_pad):
    eps = e_pad // N_SUB
    win_e = STREAMS * IDXW
    assert eps % win_e == 0
    n_win = eps // win_e
    rps = n_pad // N_SUB
    assert rps % WCHUNK == 0 and WCHUNK % 8 == 0
    packed_rows = n_pad * n_blocks * LANES // 128

    def body(z_hbm, src_hbm, dst_hbm, out_hbm,
             srcv, dstv, sidx, rows, zbuf, acc, gsem, ssem):
        zf = z_hbm
        of = out_hbm
        c = lax.axis_index("c")
        s = lax.axis_index("s")

        def fill_z(i, _):
            zbuf[i] = jnp.zeros((LANES,), jnp.float32)
            return 0
        lax.fori_loop(0, WCHUNK, fill_z, 0)

        def process(b):
            def zero(k, _):
                pltpu.sync_copy(zbuf, acc.at[pl.ds(s * rps + k * WCHUNK, WCHUNK)])
                return 0
            lax.fori_loop(0, rps // WCHUNK, zero, 0)
            plsc.subcore_barrier()
            row0 = s * (eps // IDXW)

            def win(w, _):
                r = row0 + w * STREAMS
                pltpu.sync_copy(src_hbm.at[pl.ds(r, STREAMS)], srcv)
                pltpu.sync_copy(dst_hbm.at[pl.ds(r, STREAMS)], dstv)
                for j in range(STREAMS):
                    for k in range(IDXW // LANES):
                        sl = pl.ds(k * LANES, LANES)
                        sidx[j, sl] = srcv[j, sl] * n_blocks + b
                gd = [pltpu.async_copy(zf.at[sidx.at[j]],
                                       rows.at[pl.ds(j * IDXW, IDXW)], gsem)
                      for j in range(STREAMS)]
                for d in gd:
                    d.wait()
                sd = [pltpu.async_copy(rows.at[pl.ds(j * IDXW, IDXW)],
                                       acc.at[dstv.at[j]], ssem, add=True)
                      for j in range(STREAMS)]
                for d in sd:
                    d.wait()
                return 0
            lax.fori_loop(0, n_win, win, 0)
            plsc.subcore_barrier()

            def wb(k, _):
                r0 = s * rps + k * WCHUNK
                pltpu.sync_copy(acc.at[pl.ds(r0, WCHUNK)], rows.at[pl.ds(0, WCHUNK)])
                pltpu.sync_copy(rows.at[pl.ds(0, WCHUNK)],
                                of.at[pl.ds(r0, WCHUNK), b])
                return 0
            lax.fori_loop(0, rps // WCHUNK, wb, 0)
            plsc.subcore_barrier()

        for c_val in range(N_CORES):
            blocks = list(range(c_val, n_blocks, N_CORES))
            if not blocks:
                continue

            @pl.when(c == c_val)
            def _(blocks=blocks):
                for b in blocks:
                    process(b)

    del packed_rows
    return pl.kernel(
        body,
        out_type=jax.ShapeDtypeStruct((n_pad, n_blocks, LANES), jnp.float32),
        mesh=_sc_mesh(),
        scratch_types=[
            pltpu.VMEM((STREAMS, IDXW), jnp.int32),
            pltpu.VMEM((STREAMS, IDXW), jnp.int32),
            pltpu.VMEM((STREAMS, IDXW), jnp.int32),
            pltpu.VMEM((STREAMS * IDXW, LANES), jnp.float32),
            pltpu.VMEM((WCHUNK, LANES), jnp.float32),
            pltpu.VMEM_SHARED((n_pad, LANES), jnp.float32),
            pltpu.SemaphoreType.DMA,
            pltpu.SemaphoreType.DMA,
        ],
        compiler_params=pltpu.CompilerParams(use_tc_tiling_on_sc=False),
    )


def _sc_agg(zp, src2, dst2, nb, e_pad):
    zf = zp.reshape(N_PAD * nb, LANES)
    y = _make_agg(nb, N_PAD, e_pad)(zf, src2, dst2)
    return y.reshape(N_PAD * nb * LANES // 128, 128)


GRID = N_PAD // ROW_R


def _leaky(v):
    return jnp.where(v >= 0, v, 0.01 * v)


def _pspec(d):
    rows = ROW_R * d // 128
    return pl.BlockSpec((rows, 128), lambda i: (i, 0))


def _rspec(w):
    return pl.BlockSpec((ROW_R, w), lambda i: (i, 0))


def _fspec(shape):
    return pl.BlockSpec(shape, lambda i: (0, 0))


def _packed_struct(d):
    return jax.ShapeDtypeStruct((N_PAD * d // 128, 128), jnp.float32)


def _unpack(ref, d):
    s = 128 // d
    y = ref[...]
    parts = [y[:, q * d:(q + 1) * d].reshape(ROW_R // s, 1, d) for q in range(s)]
    return jnp.concatenate(parts, axis=1).reshape(ROW_R, d)


def _pack(v, d):
    s = 128 // d
    t = v.reshape(ROW_R // s, s, d)
    return jnp.concatenate(
        [t[:, q:q + 1, :].reshape(ROW_R // s, d) for q in range(s)], axis=-1)


def _tc1(xp, W1, d0, d1):
    dW1 = W1.shape[1]

    def body(x_r, d0_r, d1_r, w_r, z_r, dinv_r):
        dinv = lax.rsqrt(d0_r[...] + d1_r[...] + 1.0)
        dinv_r[...] = dinv
        z_r[...] = _pack(dinv * jnp.dot(x_r[...], w_r[...],
                                        preferred_element_type=jnp.float32), dW1)

    return pl.pallas_call(
        body, grid=(GRID,),
        in_specs=[_rspec(xp.shape[1]), _rspec(1), _rspec(1), _fspec(W1.shape)],
        out_specs=[_pspec(dW1), _rspec(1)],
        out_shape=[_packed_struct(dW1),
                   jax.ShapeDtypeStruct((N_PAD, 1), jnp.float32)],
    )(xp, d0, d1, W1)


def _tc2(y1, z1, dinv, b1, d):
    def body(y_r, z_r, dv_r, b_r, o_r):
        h = _leaky(dv_r[...] * (_unpack(y_r, d) + _unpack(z_r, d)) + b_r[...])
        o_r[...] = _pack(dv_r[...] * h, d)

    return pl.pallas_call(
        body, grid=(GRID,),
        in_specs=[_pspec(d), _pspec(d), _rspec(1), _fspec(b1.shape)],
        out_specs=_pspec(d), out_shape=_packed_struct(d),
    )(y1, z1, dinv, b1)


def _tc3(y2, z2, dinv, W2, b2, W3, d_in):
    d_out = W3.shape[1]

    def body(y_r, z_r, dv_r, w2_r, b2_r, w3_r, o_r):
        t = dv_r[...] * (_unpack(y_r, d_in) + _unpack(z_r, d_in))
        h = _leaky(jnp.dot(t, w2_r[...], preferred_element_type=jnp.float32)
                   + b2_r[...])
        o_r[...] = _pack(dv_r[...] * jnp.dot(h, w3_r[...],
                                             preferred_element_type=jnp.float32),
                         d_out)

    return pl.pallas_call(
        body, grid=(GRID,),
        in_specs=[_pspec(d_in), _pspec(d_in), _rspec(1),
                  _fspec(W2.shape), _fspec(b2.shape), _fspec(W3.shape)],
        out_specs=_pspec(d_out), out_shape=_packed_struct(d_out),
    )(y2, z2, dinv, W2, b2, W3)


def _tc4(y3, z3, dinv, b3, W4, d_in):
    d_out = W4.shape[1]

    def body(y_r, z_r, dv_r, b3_r, w4_r, o_r):
        h = _leaky(dv_r[...] * (_unpack(y_r, d_in) + _unpack(z_r, d_in))
                   + b3_r[...])
        o_r[...] = _pack(dv_r[...] * jnp.dot(h, w4_r[...],
                                             preferred_element_type=jnp.float32),
                         d_out)

    return pl.pallas_call(
        body, grid=(GRID,),
        in_specs=[_pspec(d_in), _pspec(d_in), _rspec(1),
                  _fspec(b3.shape), _fspec(W4.shape)],
        out_specs=_pspec(d_out), out_shape=_packed_struct(d_out),
    )(y3, z3, dinv, b3, W4)


def _tc5(y4, z4, dinv, b4, Wl, bl, d_in):
    d_out = Wl.shape[1]

    def body(y_r, z_r, dv_r, b4_r, wl_r, bl_r, o_r):
        t = (dv_r[...] * (_unpack(y_r, d_in) + _unpack(z_r, d_in)))
        h = _leaky(t[:, :b4_r.shape[1]] + b4_r[...])
        o_r[...] = jnp.maximum(
            jnp.dot(h, wl_r[...], preferred_element_type=jnp.float32)
            + bl_r[...], 0.0)

    return pl.pallas_call(
        body, grid=(GRID,),
        in_specs=[_pspec(d_in), _pspec(d_in), _rspec(1),
                  _fspec(b4.shape), _fspec(Wl.shape), _fspec(bl.shape)],
        out_specs=_rspec(d_out),
        out_shape=jax.ShapeDtypeStruct((N_PAD, d_out), jnp.float32),
    )(y4, z4, dinv, b4, Wl, bl)


def kernel(x, edge_index, W1, b1, W2, b2, W3, b3, W4, b4, Wl, bl):
    n = x.shape[0]
    e = edge_index.shape[1]
    assert n <= N_PAD

    e_unit = N_CORES * N_SUB * STREAMS * IDXW
    e_pad = ((e + e_unit - 1) // e_unit) * e_unit

    src = edge_index[0].astype(jnp.int32)
    dst = edge_index[1].astype(jnp.int32)
    if e_pad != e:
        pad = e_pad - e
        pad_dst = n + jnp.arange(pad, dtype=jnp.int32) % (N_PAD - n)
        src = jnp.concatenate([src, jnp.full((pad,), n - 1, jnp.int32)])
        dst = jnp.concatenate([dst, pad_dst])
    src2 = src.reshape(e_pad // IDXW, IDXW)
    dst2 = dst.reshape(e_pad // IDXW, IDXW)

    deg0, deg1 = _make_deg(N_PAD, e_pad)(dst2)
    xp = jnp.pad(x, ((0, N_PAD - n), (0, 0)))

    b1r, b2r, b3r, b4r, blr = (v.reshape(1, -1) for v in (b1, b2, b3, b4, bl))

    W4p = jnp.pad(W4, ((0, 0), (0, 64 - W4.shape[1])))

    z1, dinv = _tc1(xp, W1, deg0[:, None], deg1[:, None])
    y1 = _sc_agg(z1, src2, dst2, 2, e_pad)
    z2 = _tc2(y1, z1, dinv, b1r, 32)
    y2 = _sc_agg(z2, src2, dst2, 2, e_pad)
    z3 = _tc3(y2, z2, dinv, W2, b2r, W3, 32)
    y3 = _sc_agg(z3, src2, dst2, 4, e_pad)
    z4 = _tc4(y3, z3, dinv, b3r, W4p, 64)
    y4 = _sc_agg(z4, src2, dst2, 4, e_pad)
    return _tc5(y4, z4, dinv, b4r, Wl, blr, 64)[:n]
